# Initial kernel scaffold; baseline (speedup 1.0000x reference)
import jax
import jax.numpy as jnp
from jax.experimental import pallas as pl


def kernel(y_true_affinity, y_pred_affinity):
    raise NotImplementedError("write your pallas kernel here")


# speedup vs baseline: 27.0389x; 27.0389x over previous
import jax
import jax.numpy as jnp
from jax import lax
from jax.experimental import pallas as pl
from jax.experimental.pallas import tpu as pltpu
from jax.experimental.pallas import tpu_sc as plsc

jax.config.update("jax_enable_x64", True)

B = 1024
T = 4096
MB = 16
L = 16
NC, NS = 2, 16
NW = NC * NS
RPW = B // NW
FB_POS = 256
FB_F = FB_POS * 2
ROW_F = 2 * T


def _row_scan(buf, idx_row, n_chunks, cnt, sm, cref, lane):
    ones = jnp.ones((L,), jnp.int32)

    def chunk(i, carry):
        @pl.when(cref[0] < MB)
        def _():
            idx = i * (2 * L) + 2 * lane
            if idx_row is None:
                a0 = plsc.load_gather(buf, [idx])
                a1 = plsc.load_gather(buf, [idx + 1])
                am = plsc.load_gather(buf, [jnp.maximum(idx - 2, 0)])
                bm = plsc.load_gather(buf, [jnp.maximum(idx - 1, 0)])
            else:
                rsel = lax.broadcast_in_dim(idx_row, (L,), ())
                a0 = plsc.load_gather(buf, [rsel, idx])
                a1 = plsc.load_gather(buf, [rsel, idx + 1])
                am = plsc.load_gather(buf, [rsel, jnp.maximum(idx - 2, 0)])
                bm = plsc.load_gather(buf, [rsel, jnp.maximum(idx - 1, 0)])
            im = (jnp.minimum(a0, a1) < 0.5).astype(jnp.int32)
            imp = (jnp.minimum(am, bm) < 0.5).astype(jnp.int32)
            first = jnp.logical_and(lane == 0, i == 0)
            imp = jnp.where(first, 0, imp)
            edge = im * (1 - imp)
            c = cref[0]
            cs = plsc.cumsum(edge) + c
            label = jnp.where(jnp.logical_and(im > 0, cs < MB), cs, 0)
            tvec = i * L + lane
            plsc.addupdate_scatter(cnt, [label], ones)
            plsc.addupdate_scatter(sm, [label], tvec)
            cref[0] = c + jnp.sum(edge, dtype=jnp.int32)
        return carry

    lax.fori_loop(jnp.int32(0), jnp.int32(n_chunks), chunk, jnp.int32(0))


def _sc_body(t_hbm, p_hbm, out_hbm, fb_t, fb_p, rest, cnt, sm, outv, cref,
             sem_a, sem_b):
    cid = lax.axis_index("c")
    sid = lax.axis_index("s")
    wid = sid * NC + cid
    base = wid * RPW
    lane = lax.iota(jnp.int32, L)

    cp_a = pltpu.async_copy(
        t_hbm.at[pl.ds(base, RPW), pl.ds(0, FB_F)], fb_t, sem_a)
    cp_b = pltpu.async_copy(
        p_hbm.at[pl.ds(base, RPW), pl.ds(0, FB_F)], fb_p, sem_b)
    cp_a.wait()
    cp_b.wait()

    outv[...] = jnp.zeros((L,), jnp.float32)

    def row_stats(fb, hbm, r):
        cnt[...] = jnp.zeros((L,), jnp.int32)
        sm[...] = jnp.zeros((L,), jnp.int32)
        cref[0] = jnp.int32(0)
        _row_scan(fb, r, FB_POS // L, cnt, sm, cref, lane)

        @pl.when(cref[0] < MB)
        def _slow():
            pltpu.sync_copy(hbm.at[base + r], rest)
            cnt[...] = jnp.zeros((L,), jnp.int32)
            sm[...] = jnp.zeros((L,), jnp.int32)
            cref[0] = jnp.int32(0)
            _row_scan(rest, None, T // L, cnt, sm, cref, lane)

        cntv = cnt[...]
        smv = sm[...]
        nb = jnp.max(jnp.where(jnp.logical_and(cntv > 0, lane >= 1), lane, 0))
        pos = smv.astype(jnp.float32) / jnp.maximum(cntv, 1).astype(jnp.float32)
        valid = jnp.logical_and(lane >= 1, lane <= nb)
        return nb, valid, pos

    def row_body(r, carry):
        nb_t, valid_t, pos_t = row_stats(fb_t, t_hbm, r)
        nb_p, valid_p, pos_p = row_stats(fb_p, p_hbm, r)
        post = jnp.where(valid_t, pos_t, jnp.float32(1e9))
        closest = jnp.full((L,), 3e9, jnp.float32)
        for j in range(1, MB):
            tj = jnp.sum(jnp.where(lane == j, post, jnp.float32(0.0)))
            closest = jnp.minimum(closest, jnp.abs(pos_p - tj))
        radius = jnp.max(jnp.where(valid_p, closest, jnp.float32(-1.0)))
        counted = jnp.logical_and(nb_t > 0, nb_p > 0)
        r_c = jnp.where(counted, radius, jnp.float32(0.0))
        n_c = jnp.where(counted, jnp.float32(1.0), jnp.float32(0.0))
        mae_c = jnp.abs(nb_t - nb_p).astype(jnp.float32)
        contrib = (jnp.where(lane == 0, mae_c, jnp.float32(0.0))
                   + jnp.where(lane == 1, r_c, jnp.float32(0.0))
                   + jnp.where(lane == 2, n_c, jnp.float32(0.0)))
        outv[...] = outv[...] + contrib
        return carry

    lax.fori_loop(jnp.int32(0), jnp.int32(RPW), row_body, jnp.int32(0))
    pltpu.sync_copy(outv, out_hbm.at[wid])


@jax.jit
def _run(t2d, p2d):
    mesh = plsc.VectorSubcoreMesh(
        core_axis_name="c", subcore_axis_name="s",
        num_cores=NC, num_subcores=NS)
    kern = pl.kernel(
        _sc_body,
        out_type=jax.ShapeDtypeStruct((NW, L), jnp.float32),
        mesh=mesh,
        compiler_params=pltpu.CompilerParams(needs_layout_passes=False),
        scratch_types=[
            pltpu.VMEM((RPW, FB_F), jnp.float32),
            pltpu.VMEM((RPW, FB_F), jnp.float32),
            pltpu.VMEM((ROW_F,), jnp.float32),
            pltpu.VMEM((L,), jnp.int32),
            pltpu.VMEM((L,), jnp.int32),
            pltpu.VMEM((L,), jnp.float32),
            pltpu.SMEM((1,), jnp.int32),
            pltpu.SemaphoreType.DMA,
            pltpu.SemaphoreType.DMA,
        ],
    )
    return kern(t2d, p2d)


def kernel(y_true_affinity, y_pred_affinity):
    t2d = y_true_affinity.reshape(B, ROW_F)
    p2d = y_pred_affinity.reshape(B, ROW_F)
    parts = _run(t2d, p2d)
    mae = jnp.sum(parts[:, 0].astype(jnp.float64))
    rsum = jnp.sum(parts[:, 1].astype(jnp.float64))
    rn = jnp.sum(parts[:, 2].astype(jnp.float64))
    n_delta = jnp.asarray(float(B), jnp.float64)
    return (mae, n_delta, rsum, rn)


# speedup vs baseline: 94.9061x; 3.5100x over previous
import jax
import jax.numpy as jnp
from jax import lax
from jax.experimental import pallas as pl
from jax.experimental.pallas import tpu as pltpu
from jax.experimental.pallas import tpu_sc as plsc

jax.config.update("jax_enable_x64", True)

B = 1024
T = 4096
MB = 16
L = 16
NC, NS = 2, 16
NW = NC * NS
RPW = B // NW
FB_POS = 256
FB_F = FB_POS * 2
ROW_F = 2 * T


def _row_scan(buf, idx_row, n_chunks, cnt, sm, cref, lane):
    ones = jnp.ones((L,), jnp.int32)

    def chunk(i, carry):
        @pl.when(cref[0] < MB)
        def _():
            t = i * L + lane
            tm = jnp.maximum(t - 1, 0)
            f0 = ((t >> 7) << 8) + (t & 127)
            fm0 = ((tm >> 7) << 8) + (tm & 127)
            if idx_row is None:
                a0 = plsc.load_gather(buf, [f0])
                a1 = plsc.load_gather(buf, [f0 + 128])
                am = plsc.load_gather(buf, [fm0])
                bm = plsc.load_gather(buf, [fm0 + 128])
            else:
                rsel = lax.broadcast_in_dim(idx_row, (L,), ())
                a0 = plsc.load_gather(buf, [rsel, f0])
                a1 = plsc.load_gather(buf, [rsel, f0 + 128])
                am = plsc.load_gather(buf, [rsel, fm0])
                bm = plsc.load_gather(buf, [rsel, fm0 + 128])
            im = (jnp.minimum(a0, a1) < 0.5).astype(jnp.int32)
            imp = (jnp.minimum(am, bm) < 0.5).astype(jnp.int32)
            first = jnp.logical_and(lane == 0, i == 0)
            imp = jnp.where(first, 0, imp)
            edge = im * (1 - imp)
            c = cref[0]
            cs = plsc.cumsum(edge) + c
            label = jnp.where(jnp.logical_and(im > 0, cs < MB), cs, 0)
            tvec = i * L + lane
            plsc.addupdate_scatter(cnt, [label], ones)
            plsc.addupdate_scatter(sm, [label], tvec)
            cref[0] = c + jnp.sum(edge, dtype=jnp.int32)
        return carry

    lax.fori_loop(jnp.int32(0), jnp.int32(n_chunks), chunk, jnp.int32(0))


def _sc_body(t_hbm, p_hbm, out_hbm, fb_t, fb_p, rest, cnt, sm, outv, cref,
             sem_a, sem_b):
    cid = lax.axis_index("c")
    sid = lax.axis_index("s")
    wid = sid * NC + cid
    base = wid * RPW
    lane = lax.iota(jnp.int32, L)

    cp_a = pltpu.async_copy(
        t_hbm.at[pl.ds(base, RPW), pl.ds(0, FB_F)], fb_t, sem_a)
    cp_b = pltpu.async_copy(
        p_hbm.at[pl.ds(base, RPW), pl.ds(0, FB_F)], fb_p, sem_b)
    cp_a.wait()
    cp_b.wait()

    outv[...] = jnp.zeros((L,), jnp.float32)

    def row_stats(fb, hbm, r):
        cnt[...] = jnp.zeros((L,), jnp.int32)
        sm[...] = jnp.zeros((L,), jnp.int32)
        cref[0] = jnp.int32(0)
        _row_scan(fb, r, FB_POS // L, cnt, sm, cref, lane)

        @pl.when(cref[0] < MB)
        def _slow():
            pltpu.sync_copy(hbm.at[base + r], rest)
            cnt[...] = jnp.zeros((L,), jnp.int32)
            sm[...] = jnp.zeros((L,), jnp.int32)
            cref[0] = jnp.int32(0)
            _row_scan(rest, None, T // L, cnt, sm, cref, lane)

        cntv = cnt[...]
        smv = sm[...]
        nb = jnp.max(jnp.where(jnp.logical_and(cntv > 0, lane >= 1), lane, 0))
        pos = smv.astype(jnp.float32) / jnp.maximum(cntv, 1).astype(jnp.float32)
        valid = jnp.logical_and(lane >= 1, lane <= nb)
        return nb, valid, pos

    def row_body(r, carry):
        nb_t, valid_t, pos_t = row_stats(fb_t, t_hbm, r)
        nb_p, valid_p, pos_p = row_stats(fb_p, p_hbm, r)
        post = jnp.where(valid_t, pos_t, jnp.float32(1e9))
        closest = jnp.full((L,), 3e9, jnp.float32)
        for j in range(1, MB):
            tj = jnp.sum(jnp.where(lane == j, post, jnp.float32(0.0)))
            closest = jnp.minimum(closest, jnp.abs(pos_p - tj))
        radius = jnp.max(jnp.where(valid_p, closest, jnp.float32(-1.0)))
        counted = jnp.logical_and(nb_t > 0, nb_p > 0)
        r_c = jnp.where(counted, radius, jnp.float32(0.0))
        n_c = jnp.where(counted, jnp.float32(1.0), jnp.float32(0.0))
        mae_c = jnp.abs(nb_t - nb_p).astype(jnp.float32)
        contrib = (jnp.where(lane == 0, mae_c, jnp.float32(0.0))
                   + jnp.where(lane == 1, r_c, jnp.float32(0.0))
                   + jnp.where(lane == 2, n_c, jnp.float32(0.0)))
        outv[...] = outv[...] + contrib
        return carry

    lax.fori_loop(jnp.int32(0), jnp.int32(RPW), row_body, jnp.int32(0))
    pltpu.sync_copy(outv, out_hbm.at[wid])


@jax.jit
def _run(t2d, p2d):
    mesh = plsc.VectorSubcoreMesh(
        core_axis_name="c", subcore_axis_name="s",
        num_cores=NC, num_subcores=NS)
    kern = pl.kernel(
        _sc_body,
        out_type=jax.ShapeDtypeStruct((NW, L), jnp.float32),
        mesh=mesh,
        compiler_params=pltpu.CompilerParams(
            needs_layout_passes=False, use_tc_tiling_on_sc=False),
        scratch_types=[
            pltpu.VMEM((RPW, FB_F), jnp.float32),
            pltpu.VMEM((RPW, FB_F), jnp.float32),
            pltpu.VMEM((ROW_F,), jnp.float32),
            pltpu.VMEM((L,), jnp.int32),
            pltpu.VMEM((L,), jnp.int32),
            pltpu.VMEM((L,), jnp.float32),
            pltpu.SMEM((1,), jnp.int32),
            pltpu.SemaphoreType.DMA,
            pltpu.SemaphoreType.DMA,
        ],
    )
    return kern(t2d, p2d)


def _native_view(x):
    return x.reshape(B, T // 128, 128, 2).transpose(0, 1, 3, 2).reshape(B, ROW_F)


def kernel(y_true_affinity, y_pred_affinity):
    t2d = _native_view(y_true_affinity)
    p2d = _native_view(y_pred_affinity)
    parts = _run(t2d, p2d)
    mae = jnp.sum(parts[:, 0].astype(jnp.float64))
    rsum = jnp.sum(parts[:, 1].astype(jnp.float64))
    rn = jnp.sum(parts[:, 2].astype(jnp.float64))
    n_delta = jnp.asarray(float(B), jnp.float64)
    return (mae, n_delta, rsum, rn)


# speedup vs baseline: 95.0745x; 1.0018x over previous
import jax
import jax.numpy as jnp
from jax import lax
from jax.experimental import pallas as pl
from jax.experimental.pallas import tpu as pltpu
from jax.experimental.pallas import tpu_sc as plsc

jax.config.update("jax_enable_x64", True)

B = 1024
T = 4096
MB = 16
L = 16
NC, NS = 2, 16
NW = NC * NS
RPW = B // NW
FB_POS = 256
FB_F = FB_POS * 2
ROW_F = 2 * T


def _row_scan(buf, idx_row, n_chunks, cnt, sm, cref, lane):
    ones = jnp.ones((L,), jnp.int32)

    def chunk(i, carry):
        @pl.when(cref[0] < MB)
        def _():
            t = i * L + lane
            tm = jnp.maximum(t - 1, 0)
            f0 = ((t >> 7) << 8) + (t & 127)
            fm0 = ((tm >> 7) << 8) + (tm & 127)
            if idx_row is None:
                a0 = plsc.load_gather(buf, [f0])
                a1 = plsc.load_gather(buf, [f0 + 128])
                am = plsc.load_gather(buf, [fm0])
                bm = plsc.load_gather(buf, [fm0 + 128])
            else:
                rsel = lax.broadcast_in_dim(idx_row, (L,), ())
                a0 = plsc.load_gather(buf, [rsel, f0])
                a1 = plsc.load_gather(buf, [rsel, f0 + 128])
                am = plsc.load_gather(buf, [rsel, fm0])
                bm = plsc.load_gather(buf, [rsel, fm0 + 128])
            im = (jnp.minimum(a0, a1) < 0.5).astype(jnp.int32)
            imp = (jnp.minimum(am, bm) < 0.5).astype(jnp.int32)
            first = jnp.logical_and(lane == 0, i == 0)
            imp = jnp.where(first, 0, imp)
            edge = im * (1 - imp)
            c = cref[0]
            cs = plsc.cumsum(edge) + c
            label = jnp.where(jnp.logical_and(im > 0, cs < MB), cs, 0)
            tvec = i * L + lane
            plsc.addupdate_scatter(cnt, [label], ones)
            plsc.addupdate_scatter(sm, [label], tvec)
            cref[0] = c + jnp.sum(edge, dtype=jnp.int32)
        return carry

    lax.fori_loop(jnp.int32(0), jnp.int32(n_chunks), chunk, jnp.int32(0))


def _sc_body(t_hbm, p_hbm, out_hbm, fb_t, fb_p, rest, cnt, sm, outv, cref,
             sem_a, sem_b):
    cid = lax.axis_index("c")
    sid = lax.axis_index("s")
    wid = sid * NC + cid
    base = wid * RPW
    lane = lax.iota(jnp.int32, L)

    cp_a = pltpu.async_copy(
        t_hbm.at[pl.ds(base, RPW), pl.ds(0, FB_F)], fb_t, sem_a)
    cp_b = pltpu.async_copy(
        p_hbm.at[pl.ds(base, RPW), pl.ds(0, FB_F)], fb_p, sem_b)
    cp_a.wait()
    cp_b.wait()

    outv[...] = jnp.zeros((L,), jnp.float32)

    def row_stats(fb, hbm, r):
        cnt[...] = jnp.zeros((L,), jnp.int32)
        sm[...] = jnp.zeros((L,), jnp.int32)
        cref[0] = jnp.int32(0)
        _row_scan(fb, r, FB_POS // L, cnt, sm, cref, lane)

        @pl.when(cref[0] < MB)
        def _slow():
            pltpu.sync_copy(hbm.at[base + r], rest)
            cnt[...] = jnp.zeros((L,), jnp.int32)
            sm[...] = jnp.zeros((L,), jnp.int32)
            cref[0] = jnp.int32(0)
            _row_scan(rest, None, T // L, cnt, sm, cref, lane)

        cntv = cnt[...]
        smv = sm[...]
        nb = jnp.max(jnp.where(jnp.logical_and(cntv > 0, lane >= 1), lane, 0))
        pos = smv.astype(jnp.float32) / jnp.maximum(cntv, 1).astype(jnp.float32)
        valid = jnp.logical_and(lane >= 1, lane <= nb)
        return nb, valid, pos

    def row_body(r, carry):
        nb_t, valid_t, pos_t = row_stats(fb_t, t_hbm, r)
        nb_p, valid_p, pos_p = row_stats(fb_p, p_hbm, r)
        post = jnp.where(valid_t, pos_t, jnp.float32(1e9))
        dn = lax.GatherDimensionNumbers(
            offset_dims=(), collapsed_slice_dims=(0,), start_index_map=(0,))
        closest = jnp.abs(pos_p - post)
        for s in range(1, MB):
            idx = (lane + s) & (L - 1)
            pr = lax.gather(post, idx[:, None], dn, (1,),
                            mode=lax.GatherScatterMode.PROMISE_IN_BOUNDS)
            closest = jnp.minimum(closest, jnp.abs(pos_p - pr))
        radius = jnp.max(jnp.where(valid_p, closest, jnp.float32(-1.0)))
        counted = jnp.logical_and(nb_t > 0, nb_p > 0)
        r_c = jnp.where(counted, radius, jnp.float32(0.0))
        n_c = jnp.where(counted, jnp.float32(1.0), jnp.float32(0.0))
        mae_c = jnp.abs(nb_t - nb_p).astype(jnp.float32)
        contrib = (jnp.where(lane == 0, mae_c, jnp.float32(0.0))
                   + jnp.where(lane == 1, r_c, jnp.float32(0.0))
                   + jnp.where(lane == 2, n_c, jnp.float32(0.0)))
        outv[...] = outv[...] + contrib
        return carry

    lax.fori_loop(jnp.int32(0), jnp.int32(RPW), row_body, jnp.int32(0))
    pltpu.sync_copy(outv, out_hbm.at[wid])


@jax.jit
def _run(t2d, p2d):
    mesh = plsc.VectorSubcoreMesh(
        core_axis_name="c", subcore_axis_name="s",
        num_cores=NC, num_subcores=NS)
    kern = pl.kernel(
        _sc_body,
        out_type=jax.ShapeDtypeStruct((NW, L), jnp.float32),
        mesh=mesh,
        compiler_params=pltpu.CompilerParams(
            needs_layout_passes=False, use_tc_tiling_on_sc=False),
        scratch_types=[
            pltpu.VMEM((RPW, FB_F), jnp.float32),
            pltpu.VMEM((RPW, FB_F), jnp.float32),
            pltpu.VMEM((ROW_F,), jnp.float32),
            pltpu.VMEM((L,), jnp.int32),
            pltpu.VMEM((L,), jnp.int32),
            pltpu.VMEM((L,), jnp.float32),
            pltpu.SMEM((1,), jnp.int32),
            pltpu.SemaphoreType.DMA,
            pltpu.SemaphoreType.DMA,
        ],
    )
    return kern(t2d, p2d)


def _native_view(x):
    return x.reshape(B, T // 128, 128, 2).transpose(0, 1, 3, 2).reshape(B, ROW_F)


def kernel(y_true_affinity, y_pred_affinity):
    t2d = _native_view(y_true_affinity)
    p2d = _native_view(y_pred_affinity)
    parts = _run(t2d, p2d)
    mae = jnp.sum(parts[:, 0].astype(jnp.float64))
    rsum = jnp.sum(parts[:, 1].astype(jnp.float64))
    rn = jnp.sum(parts[:, 2].astype(jnp.float64))
    n_delta = jnp.asarray(float(B), jnp.float64)
    return (mae, n_delta, rsum, rn)


# speedup vs baseline: 101.1784x; 1.0642x over previous
import jax
import jax.numpy as jnp
from jax import lax
from jax.experimental import pallas as pl
from jax.experimental.pallas import tpu as pltpu
from jax.experimental.pallas import tpu_sc as plsc

jax.config.update("jax_enable_x64", True)

B = 1024
T = 4096
MB = 16
L = 16
NC, NS = 2, 16
NW = NC * NS
RPW = B // NW
FB_POS = 256
FB_F = FB_POS * 2
ROW_F = 2 * T


def _row_scan(buf, idx_row, n_chunks, cnt, sm, cref, lane):
    ones = jnp.ones((L,), jnp.int32)

    def chunk(i, carry):
        @pl.when(cref[0] < MB)
        def _():
            t = i * L + lane
            tm = jnp.maximum(t - 1, 0)
            f0 = ((t >> 7) << 8) + (t & 127)
            fm0 = ((tm >> 7) << 8) + (tm & 127)
            if idx_row is None:
                a0 = plsc.load_gather(buf, [f0])
                a1 = plsc.load_gather(buf, [f0 + 128])
                am = plsc.load_gather(buf, [fm0])
                bm = plsc.load_gather(buf, [fm0 + 128])
            else:
                rsel = lax.broadcast_in_dim(idx_row, (L,), ())
                a0 = plsc.load_gather(buf, [rsel, f0])
                a1 = plsc.load_gather(buf, [rsel, f0 + 128])
                am = plsc.load_gather(buf, [rsel, fm0])
                bm = plsc.load_gather(buf, [rsel, fm0 + 128])
            im = (jnp.minimum(a0, a1) < 0.5).astype(jnp.int32)
            imp = (jnp.minimum(am, bm) < 0.5).astype(jnp.int32)
            first = jnp.logical_and(lane == 0, i == 0)
            imp = jnp.where(first, 0, imp)
            edge = im * (1 - imp)
            c = cref[0]
            cs = plsc.cumsum(edge) + c
            label = jnp.where(jnp.logical_and(im > 0, cs < MB), cs, 0)
            tvec = i * L + lane
            plsc.addupdate_scatter(cnt, [label], ones)
            plsc.addupdate_scatter(sm, [label], tvec)
            cref[0] = c + jnp.sum(edge, dtype=jnp.int32)
        return carry

    lax.fori_loop(jnp.int32(0), jnp.int32(n_chunks), chunk, jnp.int32(0))


def _sc_body(t_hbm, p_hbm, out_hbm, fb_t, fb_p, rest, cnt, sm, outv, cref,
             sem_a, sem_b):
    cid = lax.axis_index("c")
    sid = lax.axis_index("s")
    wid = sid * NC + cid
    base = wid * RPW
    lane = lax.iota(jnp.int32, L)

    cp_a = pltpu.async_copy(
        t_hbm.at[pl.ds(base, RPW), pl.ds(0, FB_F)], fb_t, sem_a)
    cp_b = pltpu.async_copy(
        p_hbm.at[pl.ds(base, RPW), pl.ds(0, FB_F)], fb_p, sem_b)
    cp_a.wait()
    cp_b.wait()

    outv[...] = jnp.zeros((L,), jnp.float32)

    def row_stats(fb, hbm, r):
        cnt[...] = jnp.zeros((L,), jnp.int32)
        sm[...] = jnp.zeros((L,), jnp.int32)
        cref[0] = jnp.int32(0)
        _row_scan(fb, r, FB_POS // L, cnt, sm, cref, lane)

        @pl.when(cref[0] < MB)
        def _slow():
            pltpu.sync_copy(hbm.at[base + r], rest)
            cnt[...] = jnp.zeros((L,), jnp.int32)
            sm[...] = jnp.zeros((L,), jnp.int32)
            cref[0] = jnp.int32(0)
            _row_scan(rest, None, T // L, cnt, sm, cref, lane)

        cntv = cnt[...]
        smv = sm[...]
        nb = jnp.max(jnp.where(jnp.logical_and(cntv > 0, lane >= 1), lane, 0))
        pos = smv.astype(jnp.float32) / jnp.maximum(cntv, 1).astype(jnp.float32)
        valid = jnp.logical_and(lane >= 1, lane <= nb)
        return nb, valid, pos

    def row_body(r, carry):
        nb_t, valid_t, pos_t = row_stats(fb_t, t_hbm, r)
        nb_p, valid_p, pos_p = row_stats(fb_p, p_hbm, r)
        post = jnp.where(valid_t, pos_t, jnp.float32(1e9))
        dn = lax.GatherDimensionNumbers(
            offset_dims=(), collapsed_slice_dims=(0,), start_index_map=(0,))
        closest = jnp.abs(pos_p - post)
        for s in range(1, MB):
            idx = (lane + s) & (L - 1)
            pr = lax.gather(post, idx[:, None], dn, (1,),
                            mode=lax.GatherScatterMode.PROMISE_IN_BOUNDS)
            closest = jnp.minimum(closest, jnp.abs(pos_p - pr))
        radius = jnp.max(jnp.where(valid_p, closest, jnp.float32(-1.0)))
        counted = jnp.logical_and(nb_t > 0, nb_p > 0)
        r_c = jnp.where(counted, radius, jnp.float32(0.0))
        n_c = jnp.where(counted, jnp.float32(1.0), jnp.float32(0.0))
        mae_c = jnp.abs(nb_t - nb_p).astype(jnp.float32)
        contrib = (jnp.where(lane == 0, mae_c, jnp.float32(0.0))
                   + jnp.where(lane == 1, r_c, jnp.float32(0.0))
                   + jnp.where(lane == 2, n_c, jnp.float32(0.0)))
        outv[...] = outv[...] + contrib
        return carry

    lax.fori_loop(jnp.int32(0), jnp.int32(RPW), row_body, jnp.int32(0))
    pltpu.sync_copy(outv, out_hbm.at[wid])


@jax.jit
def _run(t2d, p2d):
    mesh = plsc.VectorSubcoreMesh(
        core_axis_name="c", subcore_axis_name="s",
        num_cores=NC, num_subcores=NS)
    kern = pl.kernel(
        _sc_body,
        out_type=jax.ShapeDtypeStruct((NW, L), jnp.float32),
        mesh=mesh,
        compiler_params=pltpu.CompilerParams(
            needs_layout_passes=False, use_tc_tiling_on_sc=False),
        scratch_types=[
            pltpu.VMEM((RPW, FB_F), jnp.float32),
            pltpu.VMEM((RPW, FB_F), jnp.float32),
            pltpu.VMEM((ROW_F,), jnp.float32),
            pltpu.VMEM((L,), jnp.int32),
            pltpu.VMEM((L,), jnp.int32),
            pltpu.VMEM((L,), jnp.float32),
            pltpu.SMEM((1,), jnp.int32),
            pltpu.SemaphoreType.DMA,
            pltpu.SemaphoreType.DMA,
        ],
    )
    return kern(t2d, p2d)


def _native_view(x):
    return x.reshape(B, T // 128, 128, 2).transpose(0, 1, 3, 2).reshape(B, ROW_F)


def kernel(y_true_affinity, y_pred_affinity):
    t2d = _native_view(y_true_affinity)
    p2d = _native_view(y_pred_affinity)
    parts = _run(t2d, p2d)
    mae = jnp.sum(parts[:, 0]).astype(jnp.float64)
    rsum = jnp.sum(parts[:, 1]).astype(jnp.float64)
    rn = jnp.sum(parts[:, 2]).astype(jnp.float64)
    n_delta = jnp.asarray(float(B), jnp.float64)
    return (mae, n_delta, rsum, rn)


# speedup vs baseline: 153.0660x; 1.5128x over previous
import jax
import jax.numpy as jnp
from jax import lax
from jax.experimental import pallas as pl
from jax.experimental.pallas import tpu as pltpu
from jax.experimental.pallas import tpu_sc as plsc

jax.config.update("jax_enable_x64", True)

B = 1024
T = 4096
MB = 16
L = 16
NC, NS = 2, 16
NW = NC * NS
RPW = B // NW
FB_POS = 256
FB_F = FB_POS * 2
ROW_F = 2 * T


def _seg_update(t, im, imp, cnt, sm, cref, ci, ones):
    edge = im * (1 - imp)
    c = cref[ci]
    cs = plsc.cumsum(edge) + c
    label = jnp.where(jnp.logical_and(im > 0, cs < MB), cs, 0)
    plsc.addupdate_scatter(cnt, [label], ones)
    plsc.addupdate_scatter(sm, [label], t)
    cref[ci] = c + jnp.sum(edge, dtype=jnp.int32)


def _chunk_masks(vals):
    a0, a1, am, bm = vals
    im = (jnp.minimum(a0, a1) < 0.5).astype(jnp.int32)
    imp = (jnp.minimum(am, bm) < 0.5).astype(jnp.int32)
    return im, imp


def _row_scan(buf, n_chunks, cnt, sm, cref, ci, lane):
    ones = jnp.ones((L,), jnp.int32)

    def chunk(i, carry):
        @pl.when(cref[ci] < MB)
        def _():
            t = i * L + lane
            tm = jnp.maximum(t - 1, 0)
            f0 = ((t >> 7) << 8) + (t & 127)
            fm0 = ((tm >> 7) << 8) + (tm & 127)
            im, imp = _chunk_masks([plsc.load_gather(buf, [f]) for f in
                                    (f0, f0 + 128, fm0, fm0 + 128)])
            first = jnp.logical_and(lane == 0, i == 0)
            imp = jnp.where(first, 0, imp)
            _seg_update(t, im, imp, cnt, sm, cref, ci, ones)
        return carry

    lax.fori_loop(jnp.int32(0), jnp.int32(n_chunks), chunk, jnp.int32(0))


def _dual_scan(fb_t, fb_p, r, n_chunks, cnt_t, sm_t, cnt_p, sm_p, cref, lane):
    ones = jnp.ones((L,), jnp.int32)

    def chunk(i, carry):
        @pl.when(jnp.logical_or(cref[0] < MB, cref[1] < MB))
        def _():
            t = i * L + lane
            tm = jnp.maximum(t - 1, 0)
            f0 = ((t >> 7) << 8) + (t & 127)
            fm0 = ((tm >> 7) << 8) + (tm & 127)
            rsel = lax.broadcast_in_dim(r, (L,), ())
            first = jnp.logical_and(lane == 0, i == 0)
            im_t, imp_t = _chunk_masks([plsc.load_gather(fb_t, [rsel, f]) for
                                        f in (f0, f0 + 128, fm0, fm0 + 128)])
            im_p, imp_p = _chunk_masks([plsc.load_gather(fb_p, [rsel, f]) for
                                        f in (f0, f0 + 128, fm0, fm0 + 128)])
            imp_t = jnp.where(first, 0, imp_t)
            imp_p = jnp.where(first, 0, imp_p)
            _seg_update(t, im_t, imp_t, cnt_t, sm_t, cref, 0, ones)
            _seg_update(t, im_p, imp_p, cnt_p, sm_p, cref, 1, ones)
        return carry

    lax.fori_loop(jnp.int32(0), jnp.int32(n_chunks), chunk, jnp.int32(0))


def _sc_body(t_hbm, p_hbm, out_hbm, fb_t, fb_p, rest, cnt_t, sm_t, cnt_p,
             sm_p, outv, cref, sem_a, sem_b):
    cid = lax.axis_index("c")
    sid = lax.axis_index("s")
    wid = sid * NC + cid
    base = wid * RPW
    lane = lax.iota(jnp.int32, L)
    zeros = jnp.zeros((L,), jnp.int32)

    cp_a = pltpu.async_copy(
        t_hbm.at[pl.ds(base, RPW), pl.ds(0, FB_F)], fb_t, sem_a)
    cp_b = pltpu.async_copy(
        p_hbm.at[pl.ds(base, RPW), pl.ds(0, FB_F)], fb_p, sem_b)
    cp_a.wait()
    cp_b.wait()

    outv[...] = jnp.zeros((L,), jnp.float32)

    def stats_from(cnt, sm):
        cntv = cnt[...]
        smv = sm[...]
        nb = jnp.max(jnp.where(jnp.logical_and(cntv > 0, lane >= 1), lane, 0))
        pos = smv.astype(jnp.float32) / jnp.maximum(cntv, 1).astype(jnp.float32)
        valid = jnp.logical_and(lane >= 1, lane <= nb)
        return nb, valid, pos

    def row_body(r, carry):
        cnt_t[...] = zeros
        sm_t[...] = zeros
        cnt_p[...] = zeros
        sm_p[...] = zeros
        cref[0] = jnp.int32(0)
        cref[1] = jnp.int32(0)
        _dual_scan(fb_t, fb_p, r, FB_POS // L, cnt_t, sm_t, cnt_p, sm_p,
                   cref, lane)

        @pl.when(cref[0] < MB)
        def _slow_t():
            pltpu.sync_copy(t_hbm.at[base + r], rest)
            cnt_t[...] = zeros
            sm_t[...] = zeros
            cref[0] = jnp.int32(0)
            _row_scan(rest, T // L, cnt_t, sm_t, cref, 0, lane)

        @pl.when(cref[1] < MB)
        def _slow_p():
            pltpu.sync_copy(p_hbm.at[base + r], rest)
            cnt_p[...] = zeros
            sm_p[...] = zeros
            cref[1] = jnp.int32(0)
            _row_scan(rest, T // L, cnt_p, sm_p, cref, 1, lane)

        nb_t, valid_t, pos_t = stats_from(cnt_t, sm_t)
        nb_p, valid_p, pos_p = stats_from(cnt_p, sm_p)
        post = jnp.where(valid_t, pos_t, jnp.float32(1e9))
        dn = lax.GatherDimensionNumbers(
            offset_dims=(), collapsed_slice_dims=(0,), start_index_map=(0,))
        closest = jnp.abs(pos_p - post)
        for s in range(1, MB):
            idx = (lane + s) & (L - 1)
            pr = lax.gather(post, idx[:, None], dn, (1,),
                            mode=lax.GatherScatterMode.PROMISE_IN_BOUNDS)
            closest = jnp.minimum(closest, jnp.abs(pos_p - pr))
        radius = jnp.max(jnp.where(valid_p, closest, jnp.float32(-1.0)))
        counted = jnp.logical_and(nb_t > 0, nb_p > 0)
        r_c = jnp.where(counted, radius, jnp.float32(0.0))
        n_c = jnp.where(counted, jnp.float32(1.0), jnp.float32(0.0))
        mae_c = jnp.abs(nb_t - nb_p).astype(jnp.float32)
        contrib = (jnp.where(lane == 0, mae_c, jnp.float32(0.0))
                   + jnp.where(lane == 1, r_c, jnp.float32(0.0))
                   + jnp.where(lane == 2, n_c, jnp.float32(0.0)))
        outv[...] = outv[...] + contrib
        return carry

    lax.fori_loop(jnp.int32(0), jnp.int32(RPW), row_body, jnp.int32(0))
    pltpu.sync_copy(outv, out_hbm.at[wid])


@jax.jit
def _run(t2d, p2d):
    mesh = plsc.VectorSubcoreMesh(
        core_axis_name="c", subcore_axis_name="s",
        num_cores=NC, num_subcores=NS)
    kern = pl.kernel(
        _sc_body,
        out_type=jax.ShapeDtypeStruct((NW, L), jnp.float32),
        mesh=mesh,
        compiler_params=pltpu.CompilerParams(
            needs_layout_passes=False, use_tc_tiling_on_sc=False),
        scratch_types=[
            pltpu.VMEM((RPW, FB_F), jnp.float32),
            pltpu.VMEM((RPW, FB_F), jnp.float32),
            pltpu.VMEM((ROW_F,), jnp.float32),
            pltpu.VMEM((L,), jnp.int32),
            pltpu.VMEM((L,), jnp.int32),
            pltpu.VMEM((L,), jnp.int32),
            pltpu.VMEM((L,), jnp.int32),
            pltpu.VMEM((L,), jnp.float32),
            pltpu.SMEM((2,), jnp.int32),
            pltpu.SemaphoreType.DMA,
            pltpu.SemaphoreType.DMA,
        ],
    )
    return kern(t2d, p2d)


def _native_view(x):
    return x.reshape(B, T // 128, 128, 2).transpose(0, 1, 3, 2).reshape(B, ROW_F)


def kernel(y_true_affinity, y_pred_affinity):
    t2d = _native_view(y_true_affinity)
    p2d = _native_view(y_pred_affinity)
    parts = _run(t2d, p2d)
    mae = jnp.sum(parts[:, 0]).astype(jnp.float64)
    rsum = jnp.sum(parts[:, 1]).astype(jnp.float64)
    rn = jnp.sum(parts[:, 2]).astype(jnp.float64)
    n_delta = jnp.asarray(float(B), jnp.float64)
    return (mae, n_delta, rsum, rn)


# speedup vs baseline: 158.3594x; 1.0346x over previous
import jax
import jax.numpy as jnp
from jax import lax
from jax.experimental import pallas as pl
from jax.experimental.pallas import tpu as pltpu
from jax.experimental.pallas import tpu_sc as plsc

jax.config.update("jax_enable_x64", True)

B = 1024
T = 4096
MB = 16
L = 16
NC, NS = 2, 16
NW = NC * NS
RPW = B // NW
FB_POS = 256
FB_F = FB_POS * 2
ROW_F = 2 * T


def _seg_update(t, im, imp, cnt, sm, cref, ci, ones):
    edge = im * (1 - imp)
    c = cref[ci]
    cs = plsc.cumsum(edge) + c
    label = jnp.where(jnp.logical_and(im > 0, cs < MB), cs, 0)
    plsc.addupdate_scatter(cnt, [label], ones)
    plsc.addupdate_scatter(sm, [label], t)
    cref[ci] = c + jnp.sum(edge, dtype=jnp.int32)


def _chunk_masks(vals):
    a0, a1, am, bm = vals
    im = (jnp.minimum(a0, a1) < 0.5).astype(jnp.int32)
    imp = (jnp.minimum(am, bm) < 0.5).astype(jnp.int32)
    return im, imp


def _row_scan(buf, n_chunks, cnt, sm, cref, ci, lane):
    ones = jnp.ones((L,), jnp.int32)

    def chunk(i, carry):
        @pl.when(cref[ci] < MB)
        def _():
            t = i * L + lane
            tm = jnp.maximum(t - 1, 0)
            f0 = ((t >> 7) << 8) + (t & 127)
            fm0 = ((tm >> 7) << 8) + (tm & 127)
            im, imp = _chunk_masks([plsc.load_gather(buf, [f]) for f in
                                    (f0, f0 + 128, fm0, fm0 + 128)])
            first = jnp.logical_and(lane == 0, i == 0)
            imp = jnp.where(first, 0, imp)
            _seg_update(t, im, imp, cnt, sm, cref, ci, ones)
        return carry

    lax.fori_loop(jnp.int32(0), jnp.int32(n_chunks), chunk, jnp.int32(0))


def _dual_scan(fb_t, fb_p, r, n_chunks, cnt_t, sm_t, cnt_p, sm_p, cref, lane):
    ones = jnp.ones((L,), jnp.int32)
    rsel = lax.broadcast_in_dim(r, (L,), ())
    UNROLL = 2

    def subchunk(t, first_flag):
        tm = jnp.maximum(t - 1, 0)
        f0 = ((t >> 7) << 8) + (t & 127)
        fm0 = ((tm >> 7) << 8) + (tm & 127)
        im_t, imp_t = _chunk_masks([plsc.load_gather(fb_t, [rsel, f]) for
                                    f in (f0, f0 + 128, fm0, fm0 + 128)])
        im_p, imp_p = _chunk_masks([plsc.load_gather(fb_p, [rsel, f]) for
                                    f in (f0, f0 + 128, fm0, fm0 + 128)])
        imp_t = jnp.where(first_flag, 0, imp_t)
        imp_p = jnp.where(first_flag, 0, imp_p)
        _seg_update(t, im_t, imp_t, cnt_t, sm_t, cref, 0, ones)
        _seg_update(t, im_p, imp_p, cnt_p, sm_p, cref, 1, ones)

    def chunk(i, carry):
        @pl.when(jnp.logical_or(cref[0] < MB, cref[1] < MB))
        def _():
            for u in range(UNROLL):
                t = (i * UNROLL + u) * L + lane
                first = (jnp.logical_and(lane == 0, i == 0) if u == 0
                         else jnp.zeros((L,), jnp.bool_))
                subchunk(t, first)
        return carry

    lax.fori_loop(jnp.int32(0), jnp.int32(n_chunks // UNROLL), chunk,
                  jnp.int32(0))


def _sc_body(t_hbm, p_hbm, out_hbm, fb_t, fb_p, rest, cnt_t, sm_t, cnt_p,
             sm_p, outv, cref, sem_a, sem_b):
    cid = lax.axis_index("c")
    sid = lax.axis_index("s")
    wid = sid * NC + cid
    base = wid * RPW
    lane = lax.iota(jnp.int32, L)
    zeros = jnp.zeros((L,), jnp.int32)

    cp_a = pltpu.async_copy(
        t_hbm.at[pl.ds(base, RPW), pl.ds(0, FB_F)], fb_t, sem_a)
    cp_b = pltpu.async_copy(
        p_hbm.at[pl.ds(base, RPW), pl.ds(0, FB_F)], fb_p, sem_b)
    cp_a.wait()
    cp_b.wait()

    outv[...] = jnp.zeros((L,), jnp.float32)

    def stats_from(cnt, sm):
        cntv = cnt[...]
        smv = sm[...]
        nb = jnp.max(jnp.where(jnp.logical_and(cntv > 0, lane >= 1), lane, 0))
        pos = smv.astype(jnp.float32) / jnp.maximum(cntv, 1).astype(jnp.float32)
        valid = jnp.logical_and(lane >= 1, lane <= nb)
        return nb, valid, pos

    def row_body(r, carry):
        cnt_t[...] = zeros
        sm_t[...] = zeros
        cnt_p[...] = zeros
        sm_p[...] = zeros
        cref[0] = jnp.int32(0)
        cref[1] = jnp.int32(0)
        _dual_scan(fb_t, fb_p, r, FB_POS // L, cnt_t, sm_t, cnt_p, sm_p,
                   cref, lane)

        @pl.when(cref[0] < MB)
        def _slow_t():
            pltpu.sync_copy(t_hbm.at[base + r], rest)
            cnt_t[...] = zeros
            sm_t[...] = zeros
            cref[0] = jnp.int32(0)
            _row_scan(rest, T // L, cnt_t, sm_t, cref, 0, lane)

        @pl.when(cref[1] < MB)
        def _slow_p():
            pltpu.sync_copy(p_hbm.at[base + r], rest)
            cnt_p[...] = zeros
            sm_p[...] = zeros
            cref[1] = jnp.int32(0)
            _row_scan(rest, T // L, cnt_p, sm_p, cref, 1, lane)

        nb_t, valid_t, pos_t = stats_from(cnt_t, sm_t)
        nb_p, valid_p, pos_p = stats_from(cnt_p, sm_p)
        post = jnp.where(valid_t, pos_t, jnp.float32(1e9))
        dn = lax.GatherDimensionNumbers(
            offset_dims=(), collapsed_slice_dims=(0,), start_index_map=(0,))
        closest = jnp.abs(pos_p - post)
        for s in range(1, MB):
            idx = (lane + s) & (L - 1)
            pr = lax.gather(post, idx[:, None], dn, (1,),
                            mode=lax.GatherScatterMode.PROMISE_IN_BOUNDS)
            closest = jnp.minimum(closest, jnp.abs(pos_p - pr))
        radius = jnp.max(jnp.where(valid_p, closest, jnp.float32(-1.0)))
        counted = jnp.logical_and(nb_t > 0, nb_p > 0)
        r_c = jnp.where(counted, radius, jnp.float32(0.0))
        n_c = jnp.where(counted, jnp.float32(1.0), jnp.float32(0.0))
        mae_c = jnp.abs(nb_t - nb_p).astype(jnp.float32)
        contrib = (jnp.where(lane == 0, mae_c, jnp.float32(0.0))
                   + jnp.where(lane == 1, r_c, jnp.float32(0.0))
                   + jnp.where(lane == 2, n_c, jnp.float32(0.0)))
        outv[...] = outv[...] + contrib
        return carry

    lax.fori_loop(jnp.int32(0), jnp.int32(RPW), row_body, jnp.int32(0))
    pltpu.sync_copy(outv, out_hbm.at[wid])


@jax.jit
def _run(t2d, p2d):
    mesh = plsc.VectorSubcoreMesh(
        core_axis_name="c", subcore_axis_name="s",
        num_cores=NC, num_subcores=NS)
    kern = pl.kernel(
        _sc_body,
        out_type=jax.ShapeDtypeStruct((NW, L), jnp.float32),
        mesh=mesh,
        compiler_params=pltpu.CompilerParams(
            needs_layout_passes=False, use_tc_tiling_on_sc=False),
        scratch_types=[
            pltpu.VMEM((RPW, FB_F), jnp.float32),
            pltpu.VMEM((RPW, FB_F), jnp.float32),
            pltpu.VMEM((ROW_F,), jnp.float32),
            pltpu.VMEM((L,), jnp.int32),
            pltpu.VMEM((L,), jnp.int32),
            pltpu.VMEM((L,), jnp.int32),
            pltpu.VMEM((L,), jnp.int32),
            pltpu.VMEM((L,), jnp.float32),
            pltpu.SMEM((2,), jnp.int32),
            pltpu.SemaphoreType.DMA,
            pltpu.SemaphoreType.DMA,
        ],
    )
    return kern(t2d, p2d)


def _native_view(x):
    return x.reshape(B, T // 128, 128, 2).transpose(0, 1, 3, 2).reshape(B, ROW_F)


def kernel(y_true_affinity, y_pred_affinity):
    t2d = _native_view(y_true_affinity)
    p2d = _native_view(y_pred_affinity)
    parts = _run(t2d, p2d)
    packed = jnp.sum(parts[:, :3], axis=0).astype(jnp.float64)
    n_delta = jnp.asarray(float(B), jnp.float64)
    return (packed[0], n_delta, packed[1], packed[2])
